# Initial kernel scaffold; baseline (speedup 1.0000x reference)
import jax
import jax.numpy as jnp
from jax.experimental import pallas as pl


def kernel(xyzs, original_xyzs, features, original_features, W_temporal, bn_gamma, bn_beta, W_spatial):
    raise NotImplementedError("write your pallas kernel here")


# speedup vs baseline: 32.7289x; 32.7289x over previous
import functools

import jax
import jax.numpy as jnp
from jax import lax
from jax.experimental import pallas as pl
from jax.experimental.pallas import tpu as pltpu

B = 8
L2 = 3
N2 = 512
L1 = 6
N1 = 2048
IN = 256
MID = 128
OUT = 128
ORIG = 64
BN_EPS = 1e-5


def _stage1_body(feat_ref, wt_ref, wmid_ref, gamma_ref, beta_ref, h_ref):
    wt = wt_ref[0]
    s1 = jnp.zeros((MID, 1), jnp.float32)
    s2 = jnp.zeros((MID, 1), jnp.float32)
    for b in range(B):
        g = jnp.dot(wt, feat_ref[b, 0], preferred_element_type=jnp.float32)
        h_ref[0, b] = g
        s1 = s1 + jnp.sum(g, axis=1, keepdims=True)
        s2 = s2 + jnp.sum(g * g, axis=1, keepdims=True)
    inv_n = jnp.float32(1.0 / (B * N2))
    mean = s1 * inv_n
    var = s2 * inv_n - mean * mean
    rstd = lax.rsqrt(var + BN_EPS)
    scale = gamma_ref[...] * rstd
    bias = beta_ref[...] - mean * scale
    wmid = wmid_ref[...]
    for b in range(B):
        sf = jnp.maximum(h_ref[0, b] * scale + bias, 0.0)
        h_ref[0, b] = jnp.dot(wmid, sf, preferred_element_type=jnp.float32)


def _stage2_body(seed_ref, anchor_ref, h_ref, orig_ref, worig_ref, out_ref):
    s = seed_ref[0, 0]
    a = anchor_ref[0, 0]
    dx = s[:, 0:1] - a[0:1, :]
    dy = s[:, 1:2] - a[1:2, :]
    dz = s[:, 2:3] - a[2:3, :]
    d2 = (dx * dx + dy * dy) + dz * dz

    iota = lax.broadcasted_iota(jnp.int32, (N2, N1), 0)
    idxs = []
    dists = []
    for _ in range(3):
        mn = jnp.min(d2, axis=0, keepdims=True)
        sel = jnp.min(jnp.where(d2 == mn, iota, N2), axis=0, keepdims=True)
        idxs.append(sel)
        dists.append(mn)
        d2 = jnp.where(iota == sel, jnp.float32(jnp.inf), d2)

    r0 = 1.0 / (dists[0] + 1e-8)
    r1 = 1.0 / (dists[1] + 1e-8)
    r2 = 1.0 / (dists[2] + 1e-8)
    norm = (r0 + r1) + r2
    g_mat = jnp.where(iota == idxs[0], r0 / norm, 0.0)
    g_mat = g_mat + jnp.where(iota == idxs[1], r1 / norm, 0.0)
    g_mat = g_mat + jnp.where(iota == idxs[2], r2 / norm, 0.0)

    interp = jnp.dot(h_ref[0, 0], g_mat, preferred_element_type=jnp.float32)
    rest = jnp.dot(worig_ref[...], orig_ref[0, 0],
                   preferred_element_type=jnp.float32)
    out_ref[0, 0] = interp + rest


@jax.jit
def kernel(xyzs, original_xyzs, features, original_features, W_temporal,
           bn_gamma, bn_beta, W_spatial):
    w_taps = W_temporal.reshape(3, MID, IN)
    w_mid = W_spatial[:, :MID]
    w_orig = W_spatial[:, MID:]
    gamma = bn_gamma.reshape(MID, 1)
    beta = bn_beta.reshape(MID, 1)
    anchors_t = jnp.swapaxes(original_xyzs, 2, 3)

    h = pl.pallas_call(
        _stage1_body,
        grid=(L1,),
        in_specs=[
            pl.BlockSpec((B, 1, IN, N2), lambda j: (0, j // 2, 0, 0)),
            pl.BlockSpec((1, MID, IN), lambda j: (j % 2, 0, 0)),
            pl.BlockSpec((MID, MID), lambda j: (0, 0)),
            pl.BlockSpec((MID, 1), lambda j: (0, 0)),
            pl.BlockSpec((MID, 1), lambda j: (0, 0)),
        ],
        out_specs=pl.BlockSpec((1, B, MID, N2), lambda j: (j, 0, 0, 0)),
        out_shape=jax.ShapeDtypeStruct((L1, B, MID, N2), jnp.float32),
    )(features, w_taps, w_mid, gamma, beta)

    new_features = pl.pallas_call(
        _stage2_body,
        grid=(B, L1),
        in_specs=[
            pl.BlockSpec((1, 1, N2, 3), lambda b, j: (b, j // 2, 0, 0)),
            pl.BlockSpec((1, 1, 3, N1), lambda b, j: (b, j, 0, 0)),
            pl.BlockSpec((1, 1, MID, N2), lambda b, j: (j, b, 0, 0)),
            pl.BlockSpec((1, 1, ORIG, N1), lambda b, j: (b, j, 0, 0)),
            pl.BlockSpec((OUT, ORIG), lambda b, j: (0, 0)),
        ],
        out_specs=pl.BlockSpec((1, 1, OUT, N1), lambda b, j: (b, j, 0, 0)),
        out_shape=jax.ShapeDtypeStruct((B, L1, OUT, N1), jnp.float32),
    )(xyzs, anchors_t, h, original_features, w_orig)

    return original_xyzs, new_features


# speedup vs baseline: 44.1399x; 1.3487x over previous
import functools

import jax
import jax.numpy as jnp
from jax import lax
from jax.experimental import pallas as pl
from jax.experimental.pallas import tpu as pltpu

B = 8
L2 = 3
N2 = 512
L1 = 6
N1 = 2048
IN = 256
MID = 128
OUT = 128
ORIG = 64
BN_EPS = 1e-5


def _stage1_body(feat_ref, wt_ref, wmid_ref, gamma_ref, beta_ref, h_ref):
    wt = wt_ref[0]
    s1 = jnp.zeros((MID, 1), jnp.float32)
    s2 = jnp.zeros((MID, 1), jnp.float32)
    for b in range(B):
        g = jnp.dot(wt, feat_ref[b, 0], preferred_element_type=jnp.float32)
        h_ref[0, b] = g
        s1 = s1 + jnp.sum(g, axis=1, keepdims=True)
        s2 = s2 + jnp.sum(g * g, axis=1, keepdims=True)
    inv_n = jnp.float32(1.0 / (B * N2))
    mean = s1 * inv_n
    var = s2 * inv_n - mean * mean
    rstd = lax.rsqrt(var + BN_EPS)
    scale = gamma_ref[...] * rstd
    bias = beta_ref[...] - mean * scale
    wmid = wmid_ref[...]
    for b in range(B):
        sf = jnp.maximum(h_ref[0, b] * scale + bias, 0.0)
        h_ref[0, b] = jnp.dot(wmid, sf, preferred_element_type=jnp.float32)


def _stage2_body(seed_ref, anchor_ref, h_ref, orig_ref, worig_ref, out_ref):
    s = seed_ref[0, 0]
    a = anchor_ref[0, 0]
    dx = s[:, 0:1] - a[0:1, :]
    dy = s[:, 1:2] - a[1:2, :]
    dz = s[:, 2:3] - a[2:3, :]
    d2 = (dx * dx + dy * dy) + dz * dz

    g_un = jnp.zeros((N2, N1), jnp.float32)
    recips = []
    for _ in range(3):
        mn = jnp.min(d2, axis=0, keepdims=True)
        sel = d2 == mn
        r = 1.0 / (mn + 1e-8)
        recips.append(r)
        g_un = jnp.where(sel, r, g_un)
        d2 = jnp.where(sel, jnp.float32(jnp.inf), d2)

    norm = (recips[0] + recips[1]) + recips[2]
    interp = jnp.dot(h_ref[0, 0], g_un, preferred_element_type=jnp.float32)
    rest = jnp.dot(worig_ref[...], orig_ref[0, 0],
                   preferred_element_type=jnp.float32)
    out_ref[0, 0] = interp * (1.0 / norm) + rest


@jax.jit
def kernel(xyzs, original_xyzs, features, original_features, W_temporal,
           bn_gamma, bn_beta, W_spatial):
    w_taps = W_temporal.reshape(3, MID, IN)
    w_mid = W_spatial[:, :MID]
    w_orig = W_spatial[:, MID:]
    gamma = bn_gamma.reshape(MID, 1)
    beta = bn_beta.reshape(MID, 1)
    anchors_t = jnp.swapaxes(original_xyzs, 2, 3)

    h = pl.pallas_call(
        _stage1_body,
        grid=(L1,),
        in_specs=[
            pl.BlockSpec((B, 1, IN, N2), lambda j: (0, j // 2, 0, 0)),
            pl.BlockSpec((1, MID, IN), lambda j: (j % 2, 0, 0)),
            pl.BlockSpec((MID, MID), lambda j: (0, 0)),
            pl.BlockSpec((MID, 1), lambda j: (0, 0)),
            pl.BlockSpec((MID, 1), lambda j: (0, 0)),
        ],
        out_specs=pl.BlockSpec((1, B, MID, N2), lambda j: (j, 0, 0, 0)),
        out_shape=jax.ShapeDtypeStruct((L1, B, MID, N2), jnp.float32),
    )(features, w_taps, w_mid, gamma, beta)

    new_features = pl.pallas_call(
        _stage2_body,
        grid=(B, L1),
        in_specs=[
            pl.BlockSpec((1, 1, N2, 3), lambda b, j: (b, j // 2, 0, 0)),
            pl.BlockSpec((1, 1, 3, N1), lambda b, j: (b, j, 0, 0)),
            pl.BlockSpec((1, 1, MID, N2), lambda b, j: (j, b, 0, 0)),
            pl.BlockSpec((1, 1, ORIG, N1), lambda b, j: (b, j, 0, 0)),
            pl.BlockSpec((OUT, ORIG), lambda b, j: (0, 0)),
        ],
        out_specs=pl.BlockSpec((1, 1, OUT, N1), lambda b, j: (b, j, 0, 0)),
        out_shape=jax.ShapeDtypeStruct((B, L1, OUT, N1), jnp.float32),
    )(xyzs, anchors_t, h, original_features, w_orig)

    return original_xyzs, new_features
